# baseline (device time: 191395 ns/iter reference)
import os

import jax
import jax.numpy as jnp
from jax import lax
from jax.experimental import pallas as pl
from jax.experimental.pallas import tpu as pltpu

_DISABLE_RING = os.environ.get("KERNEL_DISABLE_RING") == "1"
_DISABLE_COMPUTE = os.environ.get("KERNEL_DISABLE_COMPUTE") == "1"
_DISABLE_KV = os.environ.get("KERNEL_DISABLE_KV") == "1"

N_DEV = 8
B_LOC = 2
SQ = 256
SKV = 256
HQ = 32
DH = 64
D_MODEL = 512
G = HQ // N_DEV
GD = G * DH
D_HID = N_DEV * GD


def kernel(x, Wq, K_ext, V_ext, Wo):
    def body(x_ref, wq_ref, k_hbm, v_hbm, wo_ref, out_ref,
             wq_g, wo_g, k_s, v_s,
             wq_send, wq_recv, wo_send, wo_recv, k_sems, v_sems):
        my = lax.axis_index("i")
        right = lax.rem(my + 1, N_DEV)
        left = lax.rem(my + N_DEV - 1, N_DEV)

        def kv_copies(b):
            bg = my * B_LOC + b
            ops = []
            for h in range(HQ):
                ops.append(pltpu.make_async_copy(
                    k_hbm.at[bg, :, h, :], k_s.at[b, h], k_sems.at[b, h]))
                ops.append(pltpu.make_async_copy(
                    v_hbm.at[bg, :, h, :], v_s.at[b, h], v_sems.at[b, h]))
            return ops

        if not _DISABLE_KV:
            for b in range(B_LOC):
                for op in kv_copies(b):
                    op.start()

        wq_g[pl.ds(my, 1), :, :] = wq_ref[...].astype(jnp.bfloat16)[None]
        wo_g[pl.ds(my, 1), :, :] = wo_ref[...].astype(jnp.bfloat16)[None]

        if _DISABLE_RING:
            ring_hops = 0
        else:
            ring_hops = N_DEV - 1

        barrier_sem = pltpu.get_barrier_semaphore()
        for nbr in (left, right):
            pl.semaphore_signal(barrier_sem, inc=1, device_id=(nbr,),
                                device_id_type=pl.DeviceIdType.MESH)
        pl.semaphore_wait(barrier_sem, 2)

        for h in range(ring_hops):
            src = lax.rem(my + N_DEV - h, N_DEV)
            rdmas = []
            for gref, ssem, rsem in ((wq_g, wq_send, wq_recv),
                                     (wo_g, wo_send, wo_recv)):
                rdma = pltpu.make_async_remote_copy(
                    src_ref=gref.at[src],
                    dst_ref=gref.at[src],
                    send_sem=ssem.at[h],
                    recv_sem=rsem.at[h],
                    device_id=(right,),
                    device_id_type=pl.DeviceIdType.MESH,
                )
                rdma.start()
                rdmas.append(rdma)
            for rdma in rdmas:
                rdma.wait()

        qb = lax.broadcasted_iota(jnp.int32, (SQ, SKV), 0) // 64
        kb = lax.broadcasted_iota(jnp.int32, (SQ, SKV), 1) // 64
        mask3 = (kb <= qb)[None]

        for b in range(B_LOC) if not _DISABLE_COMPUTE else ():
            xb = x_ref[b].astype(jnp.bfloat16)
            if not _DISABLE_KV:
                for op in kv_copies(b):
                    op.wait()

            acc = jnp.zeros((SQ, D_MODEL), jnp.float32)
            for j in range(N_DEV):
                qj = jnp.dot(xb, wq_g[j],
                             preferred_element_type=jnp.float32)
                qj = qj.astype(jnp.bfloat16)
                qg = jnp.stack(
                    [qj[:, hh * DH:(hh + 1) * DH] for hh in range(G)])
                kg = k_s[b, j * G:(j + 1) * G].astype(jnp.bfloat16)
                s = lax.dot_general(
                    qg, kg, (((2,), (2,)), ((0,), (0,))),
                    preferred_element_type=jnp.float32) * 0.125
                e = jnp.where(mask3, jnp.exp(s), 0.0)
                w = e * (1.0 / jnp.sum(e, axis=2, keepdims=True))
                vg = v_s[b, j * G:(j + 1) * G].astype(jnp.bfloat16)
                cg = lax.dot_general(
                    w.astype(jnp.bfloat16), vg, (((2,), (1,)), ((0,), (0,))),
                    preferred_element_type=jnp.float32)
                ctxg = jnp.concatenate(
                    [cg[hh] for hh in range(G)], axis=1).astype(jnp.bfloat16)
                acc = acc + jnp.dot(ctxg, wo_g[j],
                                    preferred_element_type=jnp.float32)
            out_ref[b] = acc

        if _DISABLE_COMPUTE:
            out_ref[...] = jnp.zeros((B_LOC, SQ, D_MODEL), jnp.float32)

    return pl.pallas_call(
        body,
        out_shape=jax.ShapeDtypeStruct((B_LOC, SQ, D_MODEL), jnp.float32),
        in_specs=[
            pl.BlockSpec(memory_space=pltpu.VMEM),
            pl.BlockSpec(memory_space=pltpu.VMEM),
            pl.BlockSpec(memory_space=pl.ANY),
            pl.BlockSpec(memory_space=pl.ANY),
            pl.BlockSpec(memory_space=pltpu.VMEM),
        ],
        out_specs=pl.BlockSpec(memory_space=pltpu.VMEM),
        scratch_shapes=[
            pltpu.VMEM((N_DEV, D_MODEL, GD), jnp.bfloat16),
            pltpu.VMEM((N_DEV, GD, D_MODEL), jnp.bfloat16),
            pltpu.VMEM((B_LOC, HQ, SKV, DH), jnp.float32),
            pltpu.VMEM((B_LOC, HQ, SKV, DH), jnp.float32),
            pltpu.SemaphoreType.DMA((N_DEV - 1,)),
            pltpu.SemaphoreType.DMA((N_DEV - 1,)),
            pltpu.SemaphoreType.DMA((N_DEV - 1,)),
            pltpu.SemaphoreType.DMA((N_DEV - 1,)),
            pltpu.SemaphoreType.DMA((B_LOC, HQ)),
            pltpu.SemaphoreType.DMA((B_LOC, HQ)),
        ],
        compiler_params=pltpu.CompilerParams(collective_id=0),
    )(x, Wq, K_ext, V_ext, Wo)


# device time: 73629 ns/iter; 2.5995x vs baseline; 2.5995x over previous
import os

import jax
import jax.numpy as jnp
from jax import lax
from jax.experimental import pallas as pl
from jax.experimental.pallas import tpu as pltpu

_DISABLE_RING = os.environ.get("KERNEL_DISABLE_RING") == "1"
_DISABLE_COMPUTE = os.environ.get("KERNEL_DISABLE_COMPUTE") == "1"
_NO_SOFTMAX = os.environ.get("KERNEL_NO_SOFTMAX") == "1"
_NO_ATTN = os.environ.get("KERNEL_NO_ATTN") == "1"

N_DEV = 8
B_LOC = 2
SQ = 256
SKV = 256
HQ = 32
DH = 64
D_MODEL = 512
G = HQ // N_DEV
GD = G * DH
D_HID = N_DEV * GD


def kernel(x, Wq, K_ext, V_ext, Wo):
    my_out = lax.axis_index("i")
    k_loc = lax.dynamic_slice_in_dim(K_ext, my_out * B_LOC, B_LOC, axis=0)
    v_loc = lax.dynamic_slice_in_dim(V_ext, my_out * B_LOC, B_LOC, axis=0)
    k_loc = jnp.transpose(k_loc, (0, 2, 1, 3)).astype(jnp.bfloat16)
    v_loc = jnp.transpose(v_loc, (0, 2, 1, 3)).astype(jnp.bfloat16)

    def body(x_ref, wq_ref, k_ref, v_ref, wo_ref, out_ref,
             wq_g, wo_g, wq_send, wq_recv, wo_send, wo_recv):
        my = lax.axis_index("i")
        right = lax.rem(my + 1, N_DEV)
        left = lax.rem(my + N_DEV - 1, N_DEV)

        wq_g[:, pl.ds(my * GD, GD)] = wq_ref[...].astype(jnp.bfloat16)
        wo_g[pl.ds(my * GD, GD), :] = wo_ref[...].astype(jnp.bfloat16)

        if _DISABLE_RING:
            ring_hops = 0
        else:
            ring_hops = N_DEV - 1

        barrier_sem = pltpu.get_barrier_semaphore()
        for nbr in (left, right):
            pl.semaphore_signal(barrier_sem, inc=1, device_id=(nbr,),
                                device_id_type=pl.DeviceIdType.MESH)
        pl.semaphore_wait(barrier_sem, 2)

        for h in range(ring_hops):
            src = lax.rem(my + N_DEV - h, N_DEV)
            col = src * GD
            rdmas = []
            for sref, ssem, rsem in (
                    (wq_g.at[:, pl.ds(col, GD)], wq_send, wq_recv),
                    (wo_g.at[pl.ds(col, GD), :], wo_send, wo_recv)):
                rdma = pltpu.make_async_remote_copy(
                    src_ref=sref,
                    dst_ref=sref,
                    send_sem=ssem.at[h],
                    recv_sem=rsem.at[h],
                    device_id=(right,),
                    device_id_type=pl.DeviceIdType.MESH,
                )
                rdma.start()
                rdmas.append(rdma)
            for rdma in rdmas:
                rdma.wait()

        qb = lax.broadcasted_iota(jnp.int32, (SQ, SKV), 0) // 64
        kb = lax.broadcasted_iota(jnp.int32, (SQ, SKV), 1) // 64
        mask3 = (kb <= qb)[None]

        for b in range(B_LOC) if not _DISABLE_COMPUTE else ():
            xb = x_ref[b].astype(jnp.bfloat16)
            qall = jnp.dot(xb, wq_g[...],
                           preferred_element_type=jnp.float32)
            qall = qall.astype(jnp.bfloat16)
            ctxs = []
            for j in range(N_DEV):
                if _NO_ATTN:
                    ctxs.append(qall[:, j * GD:(j + 1) * GD])
                    continue
                qg = jnp.stack(
                    [qall[:, (j * G + hh) * DH:(j * G + hh + 1) * DH]
                     for hh in range(G)])
                kg = k_ref[b, j * G:(j + 1) * G]
                s = lax.dot_general(
                    qg, kg, (((2,), (2,)), ((0,), (0,))),
                    preferred_element_type=jnp.float32) * 0.125
                if _NO_SOFTMAX:
                    w = s
                else:
                    e = jnp.where(mask3, jnp.exp(s), 0.0)
                    w = e * (1.0 / jnp.sum(e, axis=2, keepdims=True))
                vg = v_ref[b, j * G:(j + 1) * G]
                cg = lax.dot_general(
                    w.astype(jnp.bfloat16), vg, (((2,), (1,)), ((0,), (0,))),
                    preferred_element_type=jnp.float32)
                ctxg = jnp.concatenate(
                    [cg[hh] for hh in range(G)], axis=1).astype(jnp.bfloat16)
                ctxs.append(ctxg)
            ctx_flat = jnp.concatenate(ctxs, axis=1)
            out_ref[b] = jnp.dot(ctx_flat, wo_g[...],
                                 preferred_element_type=jnp.float32)

        if _DISABLE_COMPUTE:
            out_ref[...] = jnp.zeros((B_LOC, SQ, D_MODEL), jnp.float32)

    return pl.pallas_call(
        body,
        out_shape=jax.ShapeDtypeStruct((B_LOC, SQ, D_MODEL), jnp.float32),
        in_specs=[
            pl.BlockSpec(memory_space=pltpu.VMEM),
            pl.BlockSpec(memory_space=pltpu.VMEM),
            pl.BlockSpec(memory_space=pltpu.VMEM),
            pl.BlockSpec(memory_space=pltpu.VMEM),
            pl.BlockSpec(memory_space=pltpu.VMEM),
        ],
        out_specs=pl.BlockSpec(memory_space=pltpu.VMEM),
        scratch_shapes=[
            pltpu.VMEM((D_MODEL, D_HID), jnp.bfloat16),
            pltpu.VMEM((D_HID, D_MODEL), jnp.bfloat16),
            pltpu.SemaphoreType.DMA((N_DEV - 1,)),
            pltpu.SemaphoreType.DMA((N_DEV - 1,)),
            pltpu.SemaphoreType.DMA((N_DEV - 1,)),
            pltpu.SemaphoreType.DMA((N_DEV - 1,)),
        ],
        compiler_params=pltpu.CompilerParams(collective_id=0),
    )(x, Wq, k_loc, v_loc, Wo)


# device time: 42726 ns/iter; 4.4796x vs baseline; 1.7233x over previous
import jax
import jax.numpy as jnp
from jax import lax
from jax.experimental import pallas as pl
from jax.experimental.pallas import tpu as pltpu

N_DEV = 8
B_LOC = 2
SQ = 256
SKV = 256
HQ = 32
DH = 64
D_MODEL = 512
G = HQ // N_DEV
GD = G * DH
D_HID = N_DEV * GD
R_HOPS = 4
L_HOPS = 3


def kernel(x, Wq, K_ext, V_ext, Wo):
    my0 = lax.axis_index("i")
    k_loc = lax.dynamic_slice_in_dim(K_ext, my0 * B_LOC, B_LOC, axis=0)
    v_loc = lax.dynamic_slice_in_dim(V_ext, my0 * B_LOC, B_LOC, axis=0)
    k_loc = jnp.transpose(k_loc, (0, 2, 1, 3)).astype(jnp.bfloat16)
    v_loc = jnp.transpose(v_loc, (0, 2, 1, 3)).astype(jnp.bfloat16)

    def body(x_ref, wq_ref, k_ref, v_ref, wo_ref, out_ref,
             wq_g, wo_g, sems):
        my = lax.axis_index("i")
        right = lax.rem(my + 1, N_DEV)
        left = lax.rem(my + N_DEV - 1, N_DEV)

        wq_g[:, 0:GD] = wq_ref[...].astype(jnp.bfloat16)
        wo_g[0:GD, :] = wo_ref[...].astype(jnp.bfloat16)

        barrier_sem = pltpu.get_barrier_semaphore()
        for nbr in (left, right):
            pl.semaphore_signal(barrier_sem, inc=1, device_id=(nbr,),
                                device_id_type=pl.DeviceIdType.MESH)
        pl.semaphore_wait(barrier_sem, 2)

        def hop_t(tensor, direction, h):
            if direction == 0:
                s, d, tgt, sem = h, h + 1, right, h
            else:
                s = 0 if h == 0 else N_DEV - h
                d, tgt, sem = N_DEV - 1 - h, left, R_HOPS + h
            if tensor == 0:
                src = wq_g.at[:, pl.ds(s * GD, GD)]
                dst = wq_g.at[:, pl.ds(d * GD, GD)]
            else:
                src = wo_g.at[pl.ds(s * GD, GD), :]
                dst = wo_g.at[pl.ds(d * GD, GD), :]
            rdma = pltpu.make_async_remote_copy(
                src_ref=src, dst_ref=dst,
                send_sem=sems.at[2 * tensor, sem],
                recv_sem=sems.at[2 * tensor + 1, sem],
                device_id=(tgt,), device_id_type=pl.DeviceIdType.MESH)
            rdma.start()
            return rdma

        def hop(direction, h):
            return [hop_t(0, direction, h), hop_t(1, direction, h)]

        qb = lax.broadcasted_iota(jnp.int32, (SQ, SKV), 0) // 64
        kb = lax.broadcasted_iota(jnp.int32, (SQ, SKV), 1) // 64
        mask3 = (kb <= qb)[None]

        xbs = [x_ref[b].astype(jnp.bfloat16) for b in range(B_LOC)]

        def compute_slot(r):
            g = lax.rem(my + N_DEV - r, N_DEV)
            for b in range(B_LOC):
                qj = jnp.dot(xbs[b], wq_g[:, r * GD:(r + 1) * GD],
                             preferred_element_type=jnp.float32)
                qj = qj.astype(jnp.bfloat16)
                qg = jnp.stack(
                    [qj[:, hh * DH:(hh + 1) * DH] for hh in range(G)])
                kg = k_ref[b, pl.ds(g * G, G)]
                s = lax.dot_general(
                    qg, kg, (((2,), (2,)), ((0,), (0,))),
                    preferred_element_type=jnp.float32) * 0.125
                e = jnp.where(mask3, jnp.exp(s), 0.0)
                w = e * (1.0 / jnp.sum(e, axis=2, keepdims=True))
                vg = v_ref[b, pl.ds(g * G, G)]
                cg = lax.dot_general(
                    w.astype(jnp.bfloat16), vg, (((2,), (1,)), ((0,), (0,))),
                    preferred_element_type=jnp.float32)
                ctxg = jnp.concatenate(
                    [cg[hh] for hh in range(G)], axis=1).astype(jnp.bfloat16)
                part = jnp.dot(ctxg, wo_g[r * GD:(r + 1) * GD, :],
                               preferred_element_type=jnp.float32)
                if r == 0:
                    out_ref[b] = part
                else:
                    out_ref[b] = out_ref[b] + part

        import os as _os
        if _os.environ.get("KERNEL_SERIAL") == "1":
            descs = [hop(0, 0)]
            for h in range(R_HOPS):
                for d in descs[-1]:
                    d.wait()
                if h < R_HOPS - 1:
                    descs.append(hop(0, h + 1))
            descs.append(hop(1, 0))
            for h in range(L_HOPS):
                for d in descs[-1]:
                    d.wait()
                if h < L_HOPS - 1:
                    descs.append(hop(1, h + 1))
            for r in range(N_DEV):
                compute_slot(r)
            return

        if _os.environ.get("KERNEL_COMM_FIRST") == "1":
            r_descs = [hop(0, 0)]
            l_descs = [hop(1, 0)]
            for step in range(R_HOPS):
                for d in r_descs[step]:
                    d.wait_recv()
                if step < R_HOPS - 1:
                    r_descs.append(hop(0, step + 1))
                if step < L_HOPS:
                    for d in l_descs[step]:
                        d.wait_recv()
                    if step < L_HOPS - 1:
                        l_descs.append(hop(1, step + 1))
            for r in range(N_DEV):
                compute_slot(r)
            for grp in r_descs + l_descs:
                for d in grp:
                    d.wait_send()
            return

        chains = {(t, dd): [hop_t(t, dd, 0)] for t in (0, 1) for dd in (0, 1)}
        compute_slot(0)
        for step in range(R_HOPS):
            for t in (0, 1):
                chains[(t, 0)][step].wait_recv()
                if step < R_HOPS - 1:
                    chains[(t, 0)].append(hop_t(t, 0, step + 1))
            if step < L_HOPS:
                for t in (0, 1):
                    chains[(t, 1)][step].wait_recv()
                    if step < L_HOPS - 1:
                        chains[(t, 1)].append(hop_t(t, 1, step + 1))
            compute_slot(step + 1)
            if step < L_HOPS:
                compute_slot(N_DEV - 1 - step)
        for chain in chains.values():
            for d in chain:
                d.wait_send()

    return pl.pallas_call(
        body,
        out_shape=jax.ShapeDtypeStruct((B_LOC, SQ, D_MODEL), jnp.float32),
        in_specs=[
            pl.BlockSpec(memory_space=pltpu.VMEM),
            pl.BlockSpec(memory_space=pltpu.VMEM),
            pl.BlockSpec(memory_space=pltpu.VMEM),
            pl.BlockSpec(memory_space=pltpu.VMEM),
            pl.BlockSpec(memory_space=pltpu.VMEM),
        ],
        out_specs=pl.BlockSpec(memory_space=pltpu.VMEM),
        scratch_shapes=[
            pltpu.VMEM((D_MODEL, D_HID), jnp.bfloat16),
            pltpu.VMEM((D_HID, D_MODEL), jnp.bfloat16),
            pltpu.SemaphoreType.DMA((4, R_HOPS + L_HOPS)),
        ],
        compiler_params=pltpu.CompilerParams(collective_id=0),
    )(x, Wq, k_loc, v_loc, Wo)
